# baseline (speedup 1.0000x reference)
import functools
import jax
import jax.numpy as jnp
from jax import lax
from jax.experimental import pallas as pl
from jax.experimental.pallas import tpu as pltpu
from jax.experimental.pallas import tpu_sc as plsc

N = 65536; M = 8192; K = 32; C = 512; FF = 2048; H = 8; DH = C // H; OUT = 512
CP = 16
BM = 128
BN = 2048
C2 = 2 * C

_BF = jnp.bfloat16


def _bf16_hi_bits(x):
    r = x.astype(_BF).astype(jnp.float32)
    return lax.bitcast_convert_type(r, jnp.uint32)


def _table_block(x_ref, g_ref, b_ref, vc_ref, kpw_ref, o_ref):
    x = x_ref[...]
    mu = jnp.mean(x, -1, keepdims=True)
    xc = x - mu
    var = jnp.mean(xc * xc, -1, keepdims=True)
    xn = xc * jax.lax.rsqrt(var + 1e-5) * g_ref[...] + b_ref[...]
    a = jnp.dot(vc_ref[...].astype(_BF), kpw_ref[...],
                preferred_element_type=jnp.float32)
    hc = C // 2
    xw = (_bf16_hi_bits(xn[:, :hc]) >> 16) | (
        _bf16_hi_bits(xn[:, hc:]) & jnp.uint32(0xFFFF0000))
    aw = (_bf16_hi_bits(a[:, :hc]) >> 16) | (
        _bf16_hi_bits(a[:, hc:]) & jnp.uint32(0xFFFF0000))
    o_ref[:, :hc] = lax.bitcast_convert_type(xw, jnp.int32)
    o_ref[:, hc:] = lax.bitcast_convert_type(aw, jnp.int32)


def _build_table(vf, g, b, vc_pad, kpw):
    return pl.pallas_call(
        _table_block,
        grid=(N // BN,),
        in_specs=[
            pl.BlockSpec((BN, C), lambda i: (i, 0)),
            pl.BlockSpec((1, C), lambda i: (0, 0)),
            pl.BlockSpec((1, C), lambda i: (0, 0)),
            pl.BlockSpec((BN, CP), lambda i: (i, 0)),
            pl.BlockSpec((CP, C), lambda i: (0, 0)),
        ],
        out_specs=pl.BlockSpec((BN, C), lambda i: (i, 0)),
        out_shape=jax.ShapeDtypeStruct((N, C), jnp.int32),
    )(vf, g, b, vc_pad, kpw)


_SC_CHUNK = 64


def _sc_gather(table, flat_idx):
    rows = flat_idx.shape[0]
    per_worker = rows // 32
    steps = per_worker // _SC_CHUNK
    mesh = plsc.VectorSubcoreMesh(core_axis_name="c", subcore_axis_name="s")

    @functools.partial(
        pl.kernel, mesh=mesh,
        out_type=jax.ShapeDtypeStruct((rows, C), jnp.int32),
        scratch_types=[
            pltpu.VMEM((per_worker,), jnp.int32),
            pltpu.VMEM((_SC_CHUNK, C), jnp.int32),
            pltpu.VMEM((_SC_CHUNK, C), jnp.int32),
            pltpu.SemaphoreType.DMA,
            pltpu.SemaphoreType.DMA,
        ],
    )
    def k(table_hbm, idx_hbm, out_hbm, idx_v, buf0, buf1, gs0, gs1):
        wid = lax.axis_index("s") * 2 + lax.axis_index("c")
        base = wid * per_worker
        pltpu.sync_copy(idx_hbm.at[pl.ds(base, per_worker)], idx_v)
        bufs = (buf0, buf1)
        gsems = (gs0, gs1)

        def _gather(g, s):
            return pltpu.make_async_copy(
                table_hbm.at[idx_v.at[pl.ds(g * _SC_CHUNK, _SC_CHUNK)]],
                bufs[s], gsems[s])

        def _finish(g, s):
            _gather(g, s).wait()
            pltpu.sync_copy(
                bufs[s], out_hbm.at[pl.ds(base + g * _SC_CHUNK, _SC_CHUNK)])

        _gather(0, 0).start()
        _gather(1, 1).start()

        def pair_body(p, carry):
            for s in (0, 1):
                g = p * 2 + s
                _finish(g, s)
                _gather(g + 2, s).start()
            return carry

        lax.fori_loop(0, steps // 2 - 1, pair_body, 0)
        _finish(steps - 2, 0)
        _finish(steps - 1, 1)

    return k(table, flat_idx)


def _dense_block(tab_ref, qc_ref, kpw, kpb, qpw, qpb,
                 wqT, wkT, wvT, opT, opb, l1T, l1b, l2T, l2b,
                 ln2g, ln2b, owT, ob, S_ref, ST_ref, out_ref):
    tab = tab_ref[...]
    xn = tab[:, :C]
    a3 = tab[:, C:].reshape(BM, K, C)

    qc = qc_ref[...]
    bq = (kpb[...] - jnp.dot(qc.astype(_BF), kpw[...],
                             preferred_element_type=jnp.float32)).astype(_BF)
    kf3 = xn.reshape(BM, K, C) + jax.nn.relu(a3 + bq[:, None, :])
    kf = kf3.reshape(BM * K, C)

    pooled = jnp.max(kf3, axis=1).astype(jnp.float32)
    qpos = jax.nn.relu(jnp.dot(qc.astype(_BF), qpw[...],
                               preferred_element_type=jnp.float32) + qpb[...])
    qf = (qpos + pooled).astype(_BF)

    q = (jnp.dot(qf, wqT[...], preferred_element_type=jnp.float32)
         * (DH ** -0.5)).astype(_BF)
    k = jnp.dot(kf, wkT[...], preferred_element_type=jnp.float32).astype(_BF)
    v = jnp.dot(kf, wvT[...], preferred_element_type=jnp.float32)

    e3 = k.reshape(BM, K, C) * q[:, None, :]
    scores = jnp.dot(e3.reshape(BM * K, C), S_ref[...],
                     preferred_element_type=jnp.float32)
    s3 = scores.reshape(BM, K, H)
    s3 = s3 - jnp.max(s3, axis=1, keepdims=True)
    es = jnp.exp(s3)
    attn = (es / jnp.sum(es, axis=1, keepdims=True)).astype(_BF)
    a_exp = jnp.dot(attn.reshape(BM * K, H), ST_ref[...],
                    preferred_element_type=jnp.float32)
    ctx = jnp.sum((a_exp * v).reshape(BM, K, C), axis=1)

    attend = jnp.dot(ctx.astype(_BF), opT[...],
                     preferred_element_type=jnp.float32) + opb[...]
    hdn = jax.nn.relu(jnp.dot(attend.astype(_BF), l1T[...],
                              preferred_element_type=jnp.float32) + l1b[...])
    act = jnp.dot(hdn.astype(_BF), l2T[...],
                  preferred_element_type=jnp.float32) + l2b[...]
    y = attend + act
    mu2 = jnp.mean(y, -1, keepdims=True)
    yc = y - mu2
    var2 = jnp.mean(yc * yc, -1, keepdims=True)
    nf = yc * jax.lax.rsqrt(var2 + 1e-5) * ln2g[...] + ln2b[...]
    out_ref[...] = jax.nn.relu(jnp.dot(nf.astype(_BF), owT[...],
                                       preferred_element_type=jnp.float32) + ob[...])


def _dense_call(tab_g, qc_pad, *weights):
    mc = qc_pad.shape[0]
    grid = (mc // BM,)
    row = lambda i: (i, 0)
    full = lambda i: (0, 0)
    in_specs = [
        pl.BlockSpec((BM * K, C2), row),
        pl.BlockSpec((BM, CP), row),
    ] + [pl.BlockSpec(w.shape, full) for w in weights]
    return pl.pallas_call(
        _dense_block,
        grid=grid,
        in_specs=in_specs,
        out_specs=pl.BlockSpec((BM, OUT), row),
        out_shape=jax.ShapeDtypeStruct((mc, OUT), jnp.float32),
    )(tab_g, qc_pad, *weights)


def kernel(voxel_features, voxel_coords, query_coords, key_indices, key_mask,
           ln1_g, ln1_b, q_pos_w, q_pos_b, k_pos_w, k_pos_b, in_proj_w,
           in_proj_b, out_proj_w, out_proj_b, lin1_w, lin1_b, lin2_w, lin2_b,
           ln2_g, ln2_b, out_w, out_b):
    vc_pad = jnp.pad(voxel_coords, ((0, 0), (0, CP - 3)))
    qc_pad = jnp.pad(query_coords, ((0, 0), (0, CP - 3)))
    flat_idx = key_indices.reshape(-1)

    r2 = lambda a: a.reshape(1, -1)
    bf = lambda a: a.astype(_BF)
    kpwT = bf(jnp.pad(k_pos_w, ((0, 0), (0, CP - 3))).T)

    table = _build_table(voxel_features, r2(ln1_g), r2(ln1_b), vc_pad, kpwT)

    perm = jnp.arange(C).reshape(2, C // 2).T.reshape(-1)

    head_ids = jnp.arange(C, dtype=jnp.int32) // DH
    S = (head_ids[:, None] == jnp.arange(H, dtype=jnp.int32)[None, :]).astype(_BF)
    weights = (
        kpwT[:, perm], r2(k_pos_b[perm]),
        bf(jnp.pad(q_pos_w, ((0, 0), (0, CP - 3))).T)[:, perm], r2(q_pos_b[perm]),
        bf(in_proj_w[:C].T)[perm, :], bf(in_proj_w[C:2 * C].T)[perm, :],
        bf(in_proj_w[2 * C:].T)[perm, :],
        bf(out_proj_w.T), r2(out_proj_b),
        bf(lin1_w.T), r2(lin1_b),
        bf(lin2_w.T), r2(lin2_b),
        r2(ln2_g), r2(ln2_b),
        bf(out_w.T), r2(out_b),
        S, S.T,
    )

    nch = 4
    mc = M // nch
    outs = []
    for i in range(nch):
        idx_c = lax.dynamic_slice_in_dim(flat_idx, i * mc * K, mc * K)
        tab_c = _sc_gather(table, idx_c)
        tab_v = lax.bitcast_convert_type(tab_c, _BF).reshape(mc * K, C2)
        qc_c = lax.dynamic_slice_in_dim(qc_pad, i * mc, mc)
        outs.append(_dense_call(tab_v, qc_c, *weights))
    return jnp.concatenate(outs, axis=0)


# speedup vs baseline: 5.0501x; 5.0501x over previous
import functools
import jax
import jax.numpy as jnp
from jax import lax
from jax.experimental import pallas as pl
from jax.experimental.pallas import tpu as pltpu
from jax.experimental.pallas import tpu_sc as plsc

N = 65536; M = 8192; K = 32; C = 512; FF = 2048; H = 8; DH = C // H; OUT = 512
CP = 16
BM = 128
BN = 2048
C2 = 2 * C

_BF = jnp.bfloat16


def _bf16_hi_bits(x):
    r = x.astype(_BF).astype(jnp.float32)
    return lax.bitcast_convert_type(r, jnp.uint32)


def _table_block(x_ref, g_ref, b_ref, vc_ref, kpw_ref, o_ref):
    x = x_ref[...]
    mu = jnp.mean(x, -1, keepdims=True)
    xc = x - mu
    var = jnp.mean(xc * xc, -1, keepdims=True)
    xn = xc * jax.lax.rsqrt(var + 1e-5) * g_ref[...] + b_ref[...]
    a = jnp.dot(vc_ref[...].astype(_BF), kpw_ref[...],
                preferred_element_type=jnp.float32)
    word = (_bf16_hi_bits(xn) >> 16) | (_bf16_hi_bits(a) & jnp.uint32(0xFFFF0000))
    o_ref[...] = lax.bitcast_convert_type(word, jnp.int32)


def _build_table(vf, g, b, vc_pad, kpw):
    return pl.pallas_call(
        _table_block,
        grid=(N // BN,),
        in_specs=[
            pl.BlockSpec((BN, C), lambda i: (i, 0)),
            pl.BlockSpec((1, C), lambda i: (0, 0)),
            pl.BlockSpec((1, C), lambda i: (0, 0)),
            pl.BlockSpec((BN, CP), lambda i: (i, 0)),
            pl.BlockSpec((CP, C), lambda i: (0, 0)),
        ],
        out_specs=pl.BlockSpec((BN, C), lambda i: (i, 0)),
        out_shape=jax.ShapeDtypeStruct((N, C), jnp.int32),
    )(vf, g, b, vc_pad, kpw)


_SC_CHUNK = 64


def _sc_gather(table, flat_idx):
    rows = flat_idx.shape[0]
    per_worker = rows // 32
    steps = per_worker // _SC_CHUNK
    mesh = plsc.VectorSubcoreMesh(core_axis_name="c", subcore_axis_name="s")

    @functools.partial(
        pl.kernel, mesh=mesh,
        out_type=jax.ShapeDtypeStruct((rows, C), jnp.int32),
        scratch_types=[
            pltpu.VMEM((per_worker,), jnp.int32),
            pltpu.VMEM((_SC_CHUNK, C), jnp.int32),
            pltpu.VMEM((_SC_CHUNK, C), jnp.int32),
            pltpu.SemaphoreType.DMA,
            pltpu.SemaphoreType.DMA,
        ],
    )
    def k(table_hbm, idx_hbm, out_hbm, idx_v, buf0, buf1, gs0, gs1):
        wid = lax.axis_index("s") * 2 + lax.axis_index("c")
        base = wid * per_worker
        pltpu.sync_copy(idx_hbm.at[pl.ds(base, per_worker)], idx_v)
        bufs = (buf0, buf1)
        gsems = (gs0, gs1)

        def _gather(g, s):
            return pltpu.make_async_copy(
                table_hbm.at[idx_v.at[pl.ds(g * _SC_CHUNK, _SC_CHUNK)]],
                bufs[s], gsems[s])

        def _finish(g, s):
            _gather(g, s).wait()
            pltpu.sync_copy(
                bufs[s], out_hbm.at[pl.ds(base + g * _SC_CHUNK, _SC_CHUNK)])

        _gather(0, 0).start()
        _gather(1, 1).start()

        def pair_body(p, carry):
            for s in (0, 1):
                g = p * 2 + s
                _finish(g, s)
                _gather(g + 2, s).start()
            return carry

        lax.fori_loop(0, steps // 2 - 1, pair_body, 0)
        _finish(steps - 2, 0)
        _finish(steps - 1, 1)

    return k(table, flat_idx)


def _dense_block(tab_ref, qc_ref, kpw, kpb, qpw, qpb,
                 wqT, wkT, wvT, opT, opb, l1T, l1b, l2T, l2b,
                 ln2g, ln2b, owT, ob, S_ref, ST_ref, out_ref):
    tw = lax.bitcast_convert_type(tab_ref[...], jnp.uint32)
    xn = lax.bitcast_convert_type(tw << 16, jnp.float32).astype(_BF)
    a3 = lax.bitcast_convert_type(tw & jnp.uint32(0xFFFF0000),
                                  jnp.float32).reshape(BM, K, C)

    qc = qc_ref[...]
    bq = kpb[...] - jnp.dot(qc.astype(_BF), kpw[...],
                            preferred_element_type=jnp.float32)
    kf3 = xn.reshape(BM, K, C) + jax.nn.relu(a3 + bq[:, None, :]).astype(_BF)
    kf = kf3.reshape(BM * K, C)

    pooled = jnp.max(kf3, axis=1).astype(jnp.float32)
    qpos = jax.nn.relu(jnp.dot(qc.astype(_BF), qpw[...],
                               preferred_element_type=jnp.float32) + qpb[...])
    qf = (qpos + pooled).astype(_BF)

    q = (jnp.dot(qf, wqT[...], preferred_element_type=jnp.float32)
         * (DH ** -0.5)).astype(_BF)
    k = jnp.dot(kf, wkT[...], preferred_element_type=jnp.float32).astype(_BF)
    v = jnp.dot(kf, wvT[...], preferred_element_type=jnp.float32)

    e3 = k.reshape(BM, K, C) * q[:, None, :]
    scores = jnp.dot(e3.reshape(BM * K, C), S_ref[...],
                     preferred_element_type=jnp.float32)
    s3 = scores.reshape(BM, K, H)
    s3 = s3 - jnp.max(s3, axis=1, keepdims=True)
    es = jnp.exp(s3)
    attn = (es / jnp.sum(es, axis=1, keepdims=True)).astype(_BF)
    a_exp = jnp.dot(attn.reshape(BM * K, H), ST_ref[...],
                    preferred_element_type=jnp.float32)
    ctx = jnp.sum((a_exp * v).reshape(BM, K, C), axis=1)

    attend = jnp.dot(ctx.astype(_BF), opT[...],
                     preferred_element_type=jnp.float32) + opb[...]
    hdn = jax.nn.relu(jnp.dot(attend.astype(_BF), l1T[...],
                              preferred_element_type=jnp.float32) + l1b[...])
    act = jnp.dot(hdn.astype(_BF), l2T[...],
                  preferred_element_type=jnp.float32) + l2b[...]
    y = attend + act
    mu2 = jnp.mean(y, -1, keepdims=True)
    yc = y - mu2
    var2 = jnp.mean(yc * yc, -1, keepdims=True)
    nf = yc * jax.lax.rsqrt(var2 + 1e-5) * ln2g[...] + ln2b[...]
    out_ref[...] = jax.nn.relu(jnp.dot(nf.astype(_BF), owT[...],
                                       preferred_element_type=jnp.float32) + ob[...])


def _dense_call(tab_g, qc_pad, *weights):
    mc = qc_pad.shape[0]
    grid = (mc // BM,)
    row = lambda i: (i, 0)
    full = lambda i: (0, 0)
    in_specs = [
        pl.BlockSpec((BM * K, C), row),
        pl.BlockSpec((BM, CP), row),
    ] + [pl.BlockSpec(w.shape, full) for w in weights]
    return pl.pallas_call(
        _dense_block,
        grid=grid,
        in_specs=in_specs,
        out_specs=pl.BlockSpec((BM, OUT), row),
        out_shape=jax.ShapeDtypeStruct((mc, OUT), jnp.float32),
    )(tab_g, qc_pad, *weights)


def kernel(voxel_features, voxel_coords, query_coords, key_indices, key_mask,
           ln1_g, ln1_b, q_pos_w, q_pos_b, k_pos_w, k_pos_b, in_proj_w,
           in_proj_b, out_proj_w, out_proj_b, lin1_w, lin1_b, lin2_w, lin2_b,
           ln2_g, ln2_b, out_w, out_b):
    vc_pad = jnp.pad(voxel_coords, ((0, 0), (0, CP - 3)))
    qc_pad = jnp.pad(query_coords, ((0, 0), (0, CP - 3)))
    flat_idx = key_indices.reshape(-1)

    r2 = lambda a: a.reshape(1, -1)
    bf = lambda a: a.astype(_BF)
    kpwT = bf(jnp.pad(k_pos_w, ((0, 0), (0, CP - 3))).T)

    table = _build_table(voxel_features, r2(ln1_g), r2(ln1_b), vc_pad, kpwT)

    head_ids = jnp.arange(C, dtype=jnp.int32) // DH
    S = (head_ids[:, None] == jnp.arange(H, dtype=jnp.int32)[None, :]).astype(_BF)
    weights = (
        kpwT, r2(k_pos_b),
        bf(jnp.pad(q_pos_w, ((0, 0), (0, CP - 3))).T), r2(q_pos_b),
        bf(in_proj_w[:C].T), bf(in_proj_w[C:2 * C].T),
        bf(in_proj_w[2 * C:].T),
        bf(out_proj_w.T), r2(out_proj_b),
        bf(lin1_w.T), r2(lin1_b),
        bf(lin2_w.T), r2(lin2_b),
        r2(ln2_g), r2(ln2_b),
        bf(out_w.T), r2(out_b),
        S, S.T,
    )

    nch = 4
    mc = M // nch
    outs = []
    for i in range(nch):
        idx_c = lax.dynamic_slice_in_dim(flat_idx, i * mc * K, mc * K)
        tab_c = _sc_gather(table, idx_c)
        qc_c = lax.dynamic_slice_in_dim(qc_pad, i * mc, mc)
        outs.append(_dense_call(tab_c, qc_c, *weights))
    return jnp.concatenate(outs, axis=0)


# speedup vs baseline: 5.0702x; 1.0040x over previous
import functools
import jax
import jax.numpy as jnp
from jax import lax
from jax.experimental import pallas as pl
from jax.experimental.pallas import tpu as pltpu
from jax.experimental.pallas import tpu_sc as plsc

N = 65536; M = 8192; K = 32; C = 512; FF = 2048; H = 8; DH = C // H; OUT = 512
CP = 16
BM = 128
BN = 2048
C2 = 2 * C

_BF = jnp.bfloat16


def _bf16_hi_bits(x):
    r = x.astype(_BF).astype(jnp.float32)
    return lax.bitcast_convert_type(r, jnp.uint32)


def _table_block(x_ref, g_ref, b_ref, vc_ref, kpw_ref, o_ref):
    x = x_ref[...]
    mu = jnp.mean(x, -1, keepdims=True)
    xc = x - mu
    var = jnp.mean(xc * xc, -1, keepdims=True)
    xn = xc * jax.lax.rsqrt(var + 1e-5) * g_ref[...] + b_ref[...]
    a = jnp.dot(vc_ref[...].astype(_BF), kpw_ref[...],
                preferred_element_type=jnp.float32)
    word = (_bf16_hi_bits(xn) >> 16) | (_bf16_hi_bits(a) & jnp.uint32(0xFFFF0000))
    o_ref[...] = lax.bitcast_convert_type(word, jnp.int32)


def _build_table(vf, g, b, vc_pad, kpw):
    return pl.pallas_call(
        _table_block,
        grid=(N // BN,),
        in_specs=[
            pl.BlockSpec((BN, C), lambda i: (i, 0)),
            pl.BlockSpec((1, C), lambda i: (0, 0)),
            pl.BlockSpec((1, C), lambda i: (0, 0)),
            pl.BlockSpec((BN, CP), lambda i: (i, 0)),
            pl.BlockSpec((CP, C), lambda i: (0, 0)),
        ],
        out_specs=pl.BlockSpec((BN, C), lambda i: (i, 0)),
        out_shape=jax.ShapeDtypeStruct((N, C), jnp.int32),
    )(vf, g, b, vc_pad, kpw)


_SC_CHUNK = 64


def _sc_gather(table, flat_idx):
    rows = flat_idx.shape[0]
    per_worker = rows // 32
    steps = per_worker // _SC_CHUNK
    mesh = plsc.VectorSubcoreMesh(core_axis_name="c", subcore_axis_name="s")

    @functools.partial(
        pl.kernel, mesh=mesh,
        out_type=jax.ShapeDtypeStruct((rows, C), jnp.int32),
        scratch_types=[
            pltpu.VMEM((per_worker,), jnp.int32),
            pltpu.VMEM((_SC_CHUNK, C), jnp.int32),
            pltpu.VMEM((_SC_CHUNK, C), jnp.int32),
            pltpu.SemaphoreType.DMA,
            pltpu.SemaphoreType.DMA,
        ],
    )
    def k(table_hbm, idx_hbm, out_hbm, idx_v, buf0, buf1, gs0, gs1):
        wid = lax.axis_index("s") * 2 + lax.axis_index("c")
        base = wid * per_worker
        pltpu.sync_copy(idx_hbm.at[pl.ds(base, per_worker)], idx_v)
        bufs = (buf0, buf1)
        gsems = (gs0, gs1)

        def _gather(g, s):
            return pltpu.make_async_copy(
                table_hbm.at[idx_v.at[pl.ds(g * _SC_CHUNK, _SC_CHUNK)]],
                bufs[s], gsems[s])

        def _finish(g, s):
            _gather(g, s).wait()
            pltpu.sync_copy(
                bufs[s], out_hbm.at[pl.ds(base + g * _SC_CHUNK, _SC_CHUNK)])

        _gather(0, 0).start()
        _gather(1, 1).start()

        def pair_body(p, carry):
            for s in (0, 1):
                g = p * 2 + s
                _finish(g, s)
                _gather(g + 2, s).start()
            return carry

        lax.fori_loop(0, steps // 2 - 1, pair_body, 0)
        _finish(steps - 2, 0)
        _finish(steps - 1, 1)

    return k(table, flat_idx)


def _dense_block(tab_ref, qc_ref, kpw, kpb, qpw, qpb,
                 wqT, wkT, wvT, opT, opb, l1T, l1b, l2T, l2b,
                 ln2g, ln2b, owT, ob, S_ref, ST_ref, out_ref):
    tw = lax.bitcast_convert_type(tab_ref[...], jnp.uint32)
    xn = lax.bitcast_convert_type(tw << 16, jnp.float32).astype(_BF)
    a3 = lax.bitcast_convert_type(tw & jnp.uint32(0xFFFF0000),
                                  jnp.float32).reshape(BM, K, C)

    qc = qc_ref[...]
    bq = kpb[...] - jnp.dot(qc.astype(_BF), kpw[...],
                            preferred_element_type=jnp.float32)
    kf3 = xn.reshape(BM, K, C) + jax.nn.relu(a3 + bq[:, None, :]).astype(_BF)
    kf = kf3.reshape(BM * K, C)

    pooled = jnp.max(kf3, axis=1).astype(jnp.float32)
    qpos = jax.nn.relu(jnp.dot(qc.astype(_BF), qpw[...],
                               preferred_element_type=jnp.float32) + qpb[...])
    qf = (qpos + pooled).astype(_BF)

    q = (jnp.dot(qf, wqT[...], preferred_element_type=jnp.float32)
         * (DH ** -0.5)).astype(_BF)
    k = jnp.dot(kf, wkT[...], preferred_element_type=jnp.float32).astype(_BF)
    v = jnp.dot(kf, wvT[...], preferred_element_type=jnp.float32)

    e3 = k.reshape(BM, K, C) * q[:, None, :]
    scores = jnp.dot(e3.reshape(BM * K, C), S_ref[...],
                     preferred_element_type=jnp.float32)
    s3 = scores.reshape(BM, K, H)
    s3 = s3 - jnp.max(s3, axis=1, keepdims=True)
    es = jnp.exp(s3)
    attn = (es / jnp.sum(es, axis=1, keepdims=True)).astype(_BF)
    a_exp = jnp.dot(attn.reshape(BM * K, H), ST_ref[...],
                    preferred_element_type=jnp.float32)
    ctx = jnp.sum((a_exp * v).reshape(BM, K, C), axis=1)

    attend = jnp.dot(ctx.astype(_BF), opT[...],
                     preferred_element_type=jnp.float32) + opb[...]
    hdn = jax.nn.relu(jnp.dot(attend.astype(_BF), l1T[...],
                              preferred_element_type=jnp.float32) + l1b[...])
    act = jnp.dot(hdn.astype(_BF), l2T[...],
                  preferred_element_type=jnp.float32) + l2b[...]
    y = attend + act
    mu2 = jnp.mean(y, -1, keepdims=True)
    yc = y - mu2
    var2 = jnp.mean(yc * yc, -1, keepdims=True)
    nf = yc * jax.lax.rsqrt(var2 + 1e-5) * ln2g[...] + ln2b[...]
    out_ref[...] = jax.nn.relu(jnp.dot(nf.astype(_BF), owT[...],
                                       preferred_element_type=jnp.float32) + ob[...])


def _dense_call(tab_g, qc_pad, *weights):
    mc = qc_pad.shape[0]
    grid = (mc // BM,)
    row = lambda i: (i, 0)
    full = lambda i: (0, 0)
    in_specs = [
        pl.BlockSpec((BM * K, C), row),
        pl.BlockSpec((BM, CP), row),
    ] + [pl.BlockSpec(w.shape, full) for w in weights]
    return pl.pallas_call(
        _dense_block,
        grid=grid,
        in_specs=in_specs,
        out_specs=pl.BlockSpec((BM, OUT), row),
        out_shape=jax.ShapeDtypeStruct((mc, OUT), jnp.float32),
    )(tab_g, qc_pad, *weights)


def kernel(voxel_features, voxel_coords, query_coords, key_indices, key_mask,
           ln1_g, ln1_b, q_pos_w, q_pos_b, k_pos_w, k_pos_b, in_proj_w,
           in_proj_b, out_proj_w, out_proj_b, lin1_w, lin1_b, lin2_w, lin2_b,
           ln2_g, ln2_b, out_w, out_b):
    vc_pad = jnp.pad(voxel_coords, ((0, 0), (0, CP - 3)))
    qc_pad = jnp.pad(query_coords, ((0, 0), (0, CP - 3)))
    flat_idx = key_indices.reshape(-1)

    r2 = lambda a: a.reshape(1, -1)
    bf = lambda a: a.astype(_BF)
    kpwT = bf(jnp.pad(k_pos_w, ((0, 0), (0, CP - 3))).T)

    table = _build_table(voxel_features, r2(ln1_g), r2(ln1_b), vc_pad, kpwT)

    head_ids = jnp.arange(C, dtype=jnp.int32) // DH
    S = (head_ids[:, None] == jnp.arange(H, dtype=jnp.int32)[None, :]).astype(_BF)
    weights = (
        kpwT, r2(k_pos_b),
        bf(jnp.pad(q_pos_w, ((0, 0), (0, CP - 3))).T), r2(q_pos_b),
        bf(in_proj_w[:C].T), bf(in_proj_w[C:2 * C].T),
        bf(in_proj_w[2 * C:].T),
        bf(out_proj_w.T), r2(out_proj_b),
        bf(lin1_w.T), r2(lin1_b),
        bf(lin2_w.T), r2(lin2_b),
        r2(ln2_g), r2(ln2_b),
        bf(out_w.T), r2(out_b),
        S, S.T,
    )

    nch = 8
    mc = M // nch
    outs = []
    for i in range(nch):
        idx_c = lax.dynamic_slice_in_dim(flat_idx, i * mc * K, mc * K)
        tab_c = _sc_gather(table, idx_c)
        qc_c = lax.dynamic_slice_in_dim(qc_pad, i * mc, mc)
        outs.append(_dense_call(tab_c, qc_c, *weights))
    return jnp.concatenate(outs, axis=0)


# speedup vs baseline: 5.0710x; 1.0002x over previous
import functools
import jax
import jax.numpy as jnp
from jax import lax
from jax.experimental import pallas as pl
from jax.experimental.pallas import tpu as pltpu
from jax.experimental.pallas import tpu_sc as plsc

N = 65536; M = 8192; K = 32; C = 512; FF = 2048; H = 8; DH = C // H; OUT = 512
CP = 16
BM = 128
BN = 2048

_BF = jnp.bfloat16


def _bf16_hi_bits(x):
    r = x.astype(_BF).astype(jnp.float32)
    return lax.bitcast_convert_type(r, jnp.uint32)


def _table_block(x_ref, g_ref, b_ref, vc_ref, kpw_ref, o_ref):
    x = x_ref[...]
    mu = jnp.mean(x, -1, keepdims=True)
    xc = x - mu
    var = jnp.mean(xc * xc, -1, keepdims=True)
    xn = xc * jax.lax.rsqrt(var + 1e-5) * g_ref[...] + b_ref[...]
    a = jnp.dot(vc_ref[...].astype(_BF), kpw_ref[...],
                preferred_element_type=jnp.float32)
    word = (_bf16_hi_bits(xn) >> 16) | (_bf16_hi_bits(a) & jnp.uint32(0xFFFF0000))
    o_ref[...] = lax.bitcast_convert_type(word, jnp.int32)


def _build_table(vf, g, b, vc_pad, kpw):
    return pl.pallas_call(
        _table_block,
        grid=(N // BN,),
        in_specs=[
            pl.BlockSpec((BN, C), lambda i: (i, 0)),
            pl.BlockSpec((1, C), lambda i: (0, 0)),
            pl.BlockSpec((1, C), lambda i: (0, 0)),
            pl.BlockSpec((BN, CP), lambda i: (i, 0)),
            pl.BlockSpec((CP, C), lambda i: (0, 0)),
        ],
        out_specs=pl.BlockSpec((BN, C), lambda i: (i, 0)),
        out_shape=jax.ShapeDtypeStruct((N, C), jnp.int32),
    )(vf, g, b, vc_pad, kpw)


_SC_CHUNK = 64


def _sc_gather(table, flat_idx):
    rows = flat_idx.shape[0]
    per_worker = rows // 32
    steps = per_worker // _SC_CHUNK
    mesh = plsc.VectorSubcoreMesh(core_axis_name="c", subcore_axis_name="s")

    @functools.partial(
        pl.kernel, mesh=mesh,
        out_type=jax.ShapeDtypeStruct((rows, C), jnp.int32),
        scratch_types=[
            pltpu.VMEM((per_worker,), jnp.int32),
            pltpu.VMEM((_SC_CHUNK, C), jnp.int32),
            pltpu.VMEM((_SC_CHUNK, C), jnp.int32),
            pltpu.SemaphoreType.DMA,
            pltpu.SemaphoreType.DMA,
        ],
    )
    def k(table_hbm, idx_hbm, out_hbm, idx_v, buf0, buf1, gs0, gs1):
        wid = lax.axis_index("s") * 2 + lax.axis_index("c")
        base = wid * per_worker
        pltpu.sync_copy(idx_hbm.at[pl.ds(base, per_worker)], idx_v)
        bufs = (buf0, buf1)
        gsems = (gs0, gs1)

        def _gather(g, s):
            return pltpu.make_async_copy(
                table_hbm.at[idx_v.at[pl.ds(g * _SC_CHUNK, _SC_CHUNK)]],
                bufs[s], gsems[s])

        def _finish(g, s):
            _gather(g, s).wait()
            pltpu.sync_copy(
                bufs[s], out_hbm.at[pl.ds(base + g * _SC_CHUNK, _SC_CHUNK)])

        _gather(0, 0).start()
        _gather(1, 1).start()

        def pair_body(p, carry):
            for s in (0, 1):
                g = p * 2 + s
                _finish(g, s)
                _gather(g + 2, s).start()
            return carry

        lax.fori_loop(0, steps // 2 - 1, pair_body, 0)
        _finish(steps - 2, 0)
        _finish(steps - 1, 1)

    return k(table, flat_idx)


def _dense_block(tab_ref, qc_ref, kpw, kpb, qpw, qpb,
                 wqT, wkT, wvT, opT, opb, l1T, l1b, l2T, l2b,
                 ln2g, ln2b, owT, ob, S_ref, ST_ref, out_ref):
    tw = lax.bitcast_convert_type(tab_ref[...], jnp.uint32)
    xn = lax.bitcast_convert_type(tw << 16, jnp.float32).astype(_BF)
    a3 = lax.bitcast_convert_type(tw & jnp.uint32(0xFFFF0000),
                                  jnp.float32).reshape(BM, K, C)

    qc = qc_ref[...]
    bq = kpb[...] - jnp.dot(qc.astype(_BF), kpw[...],
                            preferred_element_type=jnp.float32)
    kf3 = xn.reshape(BM, K, C) + jax.nn.relu(a3 + bq[:, None, :]).astype(_BF)
    kf = kf3.reshape(BM * K, C)

    pooled = jnp.max(kf3, axis=1).astype(jnp.float32)
    qpos = jax.nn.relu(jnp.dot(qc.astype(_BF), qpw[...],
                               preferred_element_type=jnp.float32) + qpb[...])
    qf = (qpos + pooled).astype(_BF)

    q = (jnp.dot(qf, wqT[...], preferred_element_type=jnp.float32)
         * (DH ** -0.5)).astype(_BF)
    k = jnp.dot(kf, wkT[...], preferred_element_type=jnp.float32).astype(_BF)
    v = jnp.dot(kf, wvT[...], preferred_element_type=jnp.float32)

    e3 = k.reshape(BM, K, C) * q[:, None, :]
    scores = jnp.dot(e3.reshape(BM * K, C), S_ref[...],
                     preferred_element_type=jnp.float32)
    s3 = scores.reshape(BM, K, H)
    s3 = s3 - jnp.max(s3, axis=1, keepdims=True)
    es = jnp.exp(s3)
    attn = (es / jnp.sum(es, axis=1, keepdims=True)).astype(_BF)
    a_exp = jnp.dot(attn.reshape(BM * K, H), ST_ref[...],
                    preferred_element_type=jnp.float32)
    ctx = jnp.sum((a_exp * v).reshape(BM, K, C), axis=1)

    attend = jnp.dot(ctx.astype(_BF), opT[...],
                     preferred_element_type=jnp.float32) + opb[...]
    hdn = jax.nn.relu(jnp.dot(attend.astype(_BF), l1T[...],
                              preferred_element_type=jnp.float32) + l1b[...])
    act = jnp.dot(hdn.astype(_BF), l2T[...],
                  preferred_element_type=jnp.float32) + l2b[...]
    y = attend + act
    mu2 = jnp.mean(y, -1, keepdims=True)
    yc = y - mu2
    var2 = jnp.mean(yc * yc, -1, keepdims=True)
    nf = yc * jax.lax.rsqrt(var2 + 1e-5) * ln2g[...] + ln2b[...]
    out_ref[...] = jax.nn.relu(jnp.dot(nf.astype(_BF), owT[...],
                                       preferred_element_type=jnp.float32) + ob[...])


def _dense_call(tab_g, qc_pad, *weights):
    mc = qc_pad.shape[0]
    grid = (mc // BM,)
    row = lambda i: (i, 0)
    full = lambda i: (0, 0)
    in_specs = [
        pl.BlockSpec((BM * K, C), row),
        pl.BlockSpec((BM, CP), row),
    ] + [pl.BlockSpec(w.shape, full) for w in weights]
    return pl.pallas_call(
        _dense_block,
        grid=grid,
        in_specs=in_specs,
        out_specs=pl.BlockSpec((BM, OUT), row),
        out_shape=jax.ShapeDtypeStruct((mc, OUT), jnp.float32),
    )(tab_g, qc_pad, *weights)


def kernel(voxel_features, voxel_coords, query_coords, key_indices, key_mask,
           ln1_g, ln1_b, q_pos_w, q_pos_b, k_pos_w, k_pos_b, in_proj_w,
           in_proj_b, out_proj_w, out_proj_b, lin1_w, lin1_b, lin2_w, lin2_b,
           ln2_g, ln2_b, out_w, out_b):
    vc_pad = jnp.pad(voxel_coords, ((0, 0), (0, CP - 3)))
    qc_pad = jnp.pad(query_coords, ((0, 0), (0, CP - 3)))
    flat_idx = key_indices.reshape(-1)

    r2 = lambda a: a.reshape(1, -1)
    bf = lambda a: a.astype(_BF)
    kpwT = bf(jnp.pad(k_pos_w, ((0, 0), (0, CP - 3))).T)

    table = _build_table(voxel_features, r2(ln1_g), r2(ln1_b), vc_pad, kpwT)

    head_ids = jnp.arange(C, dtype=jnp.int32) // DH
    S = (head_ids[:, None] == jnp.arange(H, dtype=jnp.int32)[None, :]).astype(_BF)
    weights = (
        kpwT, r2(k_pos_b),
        bf(jnp.pad(q_pos_w, ((0, 0), (0, CP - 3))).T), r2(q_pos_b),
        bf(in_proj_w[:C].T), bf(in_proj_w[C:2 * C].T),
        bf(in_proj_w[2 * C:].T),
        bf(out_proj_w.T), r2(out_proj_b),
        bf(lin1_w.T), r2(lin1_b),
        bf(lin2_w.T), r2(lin2_b),
        r2(ln2_g), r2(ln2_b),
        bf(out_w.T), r2(out_b),
        S, S.T,
    )

    nch = 8
    mc = M // nch
    outs = []
    for i in range(nch):
        idx_c = lax.dynamic_slice_in_dim(flat_idx, i * mc * K, mc * K)
        tab_c = _sc_gather(table, idx_c)
        qc_c = lax.dynamic_slice_in_dim(qc_pad, i * mc, mc)
        outs.append(_dense_call(tab_c, qc_c, *weights))
    return jnp.concatenate(outs, axis=0)
